# baseline (speedup 1.0000x reference)
import functools

import jax
import jax.numpy as jnp
from jax import lax
from jax.experimental import pallas as pl
from jax.experimental.pallas import tpu as pltpu
from jax.experimental.pallas import tpu_sc as plsc

_VOCAB = 1000000
_EMBED = 32
_BATCH = 4096
_HIST = 200
_TOTAL = _BATCH * _HIST

_NC = 2
_NS = 16
_NW = _NC * _NS
_ROWS_W = _BATCH // _NW
_PER_W = _ROWS_W * _HIST
_SH = 2
_CH = _SH * _ROWS_W
_NSUPER = _HIST // _SH
_NITER = _NSUPER // 2


def _sc_gather(idx_flat, table):
  mesh = plsc.VectorSubcoreMesh(core_axis_name="c", subcore_axis_name="s")

  @functools.partial(
      pl.kernel,
      mesh=mesh,
      out_type=jax.ShapeDtypeStruct((_HIST, _EMBED, _BATCH), jnp.float32),
      scratch_types=[
          pltpu.VMEM((_PER_W,), jnp.int32),
          pltpu.VMEM((_PER_W,), jnp.int32),
          pltpu.VMEM((2, _CH, _EMBED), jnp.float32),
          pltpu.VMEM((2, _SH, _EMBED, _ROWS_W), jnp.float32),
          pltpu.SemaphoreType.DMA,
          pltpu.SemaphoreType.DMA,
          pltpu.SemaphoreType.DMA,
          pltpu.SemaphoreType.DMA,
      ],
      compiler_params=pltpu.CompilerParams(
          use_tc_tiling_on_sc=False, needs_layout_passes=False),
  )
  def k(idx_hbm, table_hbm, out_hbm, idx_v, idxt_v, rows_v, trows_v,
        g0, g1, s0, s1):
    wid = lax.axis_index("s") * _NC + lax.axis_index("c")
    c0 = wid * _ROWS_W
    gsem = (g0, g1)
    ssem = (s0, s1)
    iota = lax.iota(jnp.int32, 16)

    pltpu.sync_copy(idx_hbm.at[pl.ds(c0 * _HIST, _PER_W)], idx_v)

    def build_idxt(h, carry):
      for g in range(_ROWS_W // 16):
        flat = (iota + 16 * g) * _HIST + h
        idxt_v[pl.ds(h * _ROWS_W + 16 * g, 16)] = plsc.load_gather(
            idx_v, [flat])
      return carry

    lax.fori_loop(0, _HIST, build_idxt, 0)

    def gather_start(s, b):
      return pltpu.async_copy(
          table_hbm.at[idxt_v.at[pl.ds(s * _CH, _CH)]],
          rows_v.at[b], gsem[b])

    def gather_wait(b):
      pltpu.make_async_copy(
          table_hbm.at[pl.ds(0, _CH)], rows_v.at[b], gsem[b]).wait()

    def store_start(s, b):
      return pltpu.async_copy(
          trows_v.at[b],
          out_hbm.at[pl.ds(s * _SH, _SH), :, pl.ds(c0, _ROWS_W)], ssem[b])

    def store_wait(b):
      pltpu.make_async_copy(
          trows_v.at[b],
          out_hbm.at[pl.ds(0, _SH), :, pl.ds(c0, _ROWS_W)], ssem[b]).wait()

    gather_start(0, 0)
    gather_start(1, 1)

    def body(ho, carry):
      for j in range(2):
        s = ho * 2 + j

        @pl.when(ho > 0)
        def _():
          store_wait(j)

        gather_wait(j)
        for jh in range(_SH):
          for f in range(_EMBED):
            for g in range(_ROWS_W // 16):
              ridx = iota + (jh * _ROWS_W + 16 * g)
              cidx = jnp.full((16,), f, jnp.int32)
              trows_v[j, jh, f, pl.ds(16 * g, 16)] = plsc.load_gather(
                  rows_v.at[j], [ridx, cidx])
        store_start(s, j)

        @pl.when(ho < _NITER - 1)
        def _():
          gather_start(s + 2, j)

      return carry

    lax.fori_loop(0, _NITER, body, 0)
    store_wait(0)
    store_wait(1)

  return k(idx_flat, table)


def kernel(inputs, embeddings):
  idx_flat = inputs.reshape(-1).astype(jnp.int32)
  out_t = _sc_gather(idx_flat, embeddings)
  return out_t.transpose(2, 0, 1)
